# baseline (speedup 1.0000x reference)
import functools

import numpy as np
import jax
import jax.numpy as jnp
from jax import lax
from jax.experimental import pallas as pl
from jax.experimental.pallas import tpu as pltpu
from jax.experimental.pallas import tpu_sc as plsc


def _sinusoidal_pe(seq_len, d_model):
    pos = np.arange(seq_len, dtype=np.float32)[:, None]
    div = np.exp(np.arange(0, d_model, 2, dtype=np.float32) * (-np.log(10000.0) / d_model))
    pe = np.zeros((seq_len, d_model), dtype=np.float32)
    pe[:, 0::2] = np.sin(pos * div)
    pe[:, 1::2] = np.cos(pos * div)
    return pe


def _pack_perm(e):
    h = e // 2
    j = np.arange(h)
    perm = np.empty(e, dtype=np.int64)
    perm[:h] = 32 * (j // 16) + (j % 16)
    perm[h:] = 32 * (j // 16) + 16 + (j % 16)
    return perm


def _transform(table, pe, w, b):
    v, e = table.shape
    h = e // 2
    s = pe.shape[0]
    bm = 2000
    assert v % bm == 0
    nblk = v // bm

    def body(x_ref, pe_ref, w_ref, b_ref, ow_ref, op_ref):
        pid = pl.program_id(0)

        @pl.when(pid < nblk)
        def _():
            y = jnp.dot(x_ref[...], w_ref[:e, :], preferred_element_type=jnp.float32)
            u = lax.bitcast_convert_type(y, jnp.uint32)
            bf = (u + jnp.uint32(0x7FFF) + ((u >> 16) & jnp.uint32(1))) >> 16
            ow_ref[...] = bf[:, :h] | (bf[:, h:] << 16)

        @pl.when(pid == nblk)
        def _():
            op_ref[...] = (
                jnp.dot(pe_ref[...], w_ref[e:, :], preferred_element_type=jnp.float32)
                + b_ref[...]
            )

    return pl.pallas_call(
        body,
        grid=(nblk + 1,),
        in_specs=[
            pl.BlockSpec((bm, e), lambda i: (jnp.minimum(i, nblk - 1), 0)),
            pl.BlockSpec((s, e), lambda i: (0, 0)),
            pl.BlockSpec((2 * e, e), lambda i: (0, 0)),
            pl.BlockSpec((1, e), lambda i: (0, 0)),
        ],
        out_specs=[
            pl.BlockSpec((bm, h), lambda i: (jnp.minimum(i, nblk - 1), 0)),
            pl.BlockSpec((s, e), lambda i: (0, 0)),
        ],
        out_shape=[
            jax.ShapeDtypeStruct((v, h), jnp.uint32),
            jax.ShapeDtypeStruct((s, e), jnp.float32),
        ],
    )(table, pe, w, b.reshape(1, e))


def _sc_gather(idx, table2w, pos2, B, S, E):
    info = plsc.get_sparse_core_info()
    NC, NS = info.num_cores, info.num_subcores
    NW = NC * NS
    EW = E // 2
    U = S // 2
    UP = 104
    nu = (B * S) // U
    assert nu % NW == 0
    upw = nu // NW
    bpw = B // NW

    mesh = plsc.VectorSubcoreMesh(core_axis_name="c", subcore_axis_name="s")

    @functools.partial(
        pl.kernel,
        mesh=mesh,
        compiler_params=pltpu.CompilerParams(
            needs_layout_passes=False, use_tc_tiling_on_sc=False
        ),
        out_type=jax.ShapeDtypeStruct((B * S, E), jnp.float32),
        scratch_types=[
            pltpu.VMEM((nu // NW * UP,), jnp.int32),
            pltpu.VMEM((S, E), jnp.float32),
            [pltpu.VMEM((S, EW), jnp.uint32)] * 2,
            [pltpu.VMEM((S, E), jnp.float32)] * 2,
            pltpu.SemaphoreType.DMA,
            pltpu.SemaphoreType.DMA,
        ],
    )
    def k(idx_hbm, table2_hbm, pos2_hbm, out_hbm, idx_v, pos_v, gbufs, obufs, sem_g, sem_w):
        wid = lax.axis_index("s") * NC + lax.axis_index("c")
        base_b = wid * bpw
        pltpu.sync_copy(pos2_hbm, pos_v)
        pltpu.sync_copy(idx_hbm.at[pl.ds(wid * upw * UP, upw * UP)], idx_v)

        def gather_descs(i, buf):
            return tuple(
                pltpu.make_async_copy(
                    table2_hbm.at[idx_v.at[pl.ds((2 * i + half) * UP, U)]],
                    buf.at[pl.ds(half * U, U)],
                    sem_g,
                )
                for half in range(2)
            )

        def out_desc(i, buf):
            return pltpu.make_async_copy(
                buf, out_hbm.at[pl.ds((base_b + i) * S, S)], sem_w
            )

        def expand_add(gbuf, obuf):
            def rbody(r2, c2):
                for dr in range(2):
                    r = r2 * 2 + dr
                    for c in range(EW // 16):
                        w16 = gbuf[r, pl.ds(c * 16, 16)]
                        lo = plsc.bitcast(w16 << jnp.uint32(16), jnp.float32)
                        hi = plsc.bitcast(w16 & jnp.uint32(0xFFFF0000), jnp.float32)
                        obuf[r, pl.ds(c * 32, 16)] = lo + pos_v[r, pl.ds(c * 32, 16)]
                        obuf[r, pl.ds(c * 32 + 16, 16)] = (
                            hi + pos_v[r, pl.ds(c * 32 + 16, 16)]
                        )
                return c2

            lax.fori_loop(0, S // 2, rbody, 0)

        def step(i, h, fire, guard):
            gbuf = gbufs[h]
            for d in gather_descs(i, gbuf):
                d.wait()
            if fire:
                for d in gather_descs(i + 1, gbufs[1 - h]):
                    d.start()
            if guard:

                @pl.when(i >= 2)
                def _():
                    out_desc(i - 2, obufs[h]).wait()

            elif i >= 2:
                out_desc(i - 2, obufs[h]).wait()
            expand_add(gbuf, obufs[h])
            out_desc(i, obufs[h]).start()

        for d in gather_descs(0, gbufs[0]):
            d.start()

        def body(j, carry):
            for h in range(2):
                step(j * 2 + h, h, fire=True, guard=True)
            return carry

        lax.fori_loop(0, (bpw - 2) // 2, body, 0)
        step(bpw - 2, (bpw - 2) % 2, fire=True, guard=False)
        step(bpw - 1, (bpw - 1) % 2, fire=False, guard=False)
        out_desc(bpw - 2, obufs[0]).wait()
        out_desc(bpw - 1, obufs[1]).wait()

    idx_pad = jnp.pad(idx.reshape(nu, U), ((0, 0), (0, UP - U))).reshape(nu * UP)
    return k(idx_pad, table2w, pos2)


def kernel(pre_embedding, preembed_mask, embed_table, W, b):
    B, S = pre_embedding.shape
    V, E = embed_table.shape
    perm = _pack_perm(E)
    w_perm = jnp.concatenate([W[:E, :][:, perm], W[E:, :]], axis=0)
    pe = jnp.asarray(_sinusoidal_pe(S, E))

    table2w, pos2 = _transform(embed_table, pe, w_perm, b)
    idx = pre_embedding.astype(jnp.int32)
    out = _sc_gather(idx, table2w, pos2, B, S, E)
    return (out.reshape(B, S, E), preembed_mask)


# speedup vs baseline: 1.8607x; 1.8607x over previous
import functools

import numpy as np
import jax
import jax.numpy as jnp
from jax import lax
from jax.experimental import pallas as pl
from jax.experimental.pallas import tpu as pltpu
from jax.experimental.pallas import tpu_sc as plsc


def _sinusoidal_pe(seq_len, d_model):
    pos = np.arange(seq_len, dtype=np.float32)[:, None]
    div = np.exp(np.arange(0, d_model, 2, dtype=np.float32) * (-np.log(10000.0) / d_model))
    pe = np.zeros((seq_len, d_model), dtype=np.float32)
    pe[:, 0::2] = np.sin(pos * div)
    pe[:, 1::2] = np.cos(pos * div)
    return pe


def _transform(table, pe, w, b):
    v, e = table.shape
    s = pe.shape[0]
    bm = 2000
    assert v % bm == 0
    nblk = v // bm

    def body(x_ref, pe_ref, w_ref, b_ref, o_ref):
        pid = pl.program_id(0)

        @pl.when(pid < nblk)
        def _():
            o_ref[...] = jnp.dot(
                x_ref[...], w_ref[:e, :], preferred_element_type=jnp.float32
            )

        @pl.when(pid == nblk)
        def _():
            o_ref[:s, :] = (
                jnp.dot(pe_ref[...], w_ref[e:, :], preferred_element_type=jnp.float32)
                + b_ref[...]
            )

    return pl.pallas_call(
        body,
        grid=(nblk + 1,),
        in_specs=[
            pl.BlockSpec((bm, e), lambda i: (jnp.minimum(i, nblk - 1), 0)),
            pl.BlockSpec((s, e), lambda i: (0, 0)),
            pl.BlockSpec((2 * e, e), lambda i: (0, 0)),
            pl.BlockSpec((1, e), lambda i: (0, 0)),
        ],
        out_specs=pl.BlockSpec((bm, e), lambda i: (i, 0)),
        out_shape=jax.ShapeDtypeStruct((v + bm, e), jnp.float32),
    )(table, pe, w, b.reshape(1, e))


def _sc_gather(idx, table2, V, B, S, E):
    info = plsc.get_sparse_core_info()
    NC, NS = info.num_cores, info.num_subcores
    NW = NC * NS
    U = S // 2
    UP = 104
    nu = (B * S) // U
    assert nu % NW == 0
    upw = nu // NW
    NBB = 3
    bpw = B // NW
    T = upw

    mesh = plsc.VectorSubcoreMesh(core_axis_name="c", subcore_axis_name="s")

    @functools.partial(
        pl.kernel,
        mesh=mesh,
        out_type=jax.ShapeDtypeStruct((B * S, E), jnp.float32),
        scratch_types=[
            pltpu.VMEM((nu // NW * UP,), jnp.int32),
            pltpu.VMEM((S, E), jnp.float32),
            [pltpu.VMEM((S, E), jnp.float32)] * 3,
            pltpu.SemaphoreType.DMA,
            pltpu.SemaphoreType.DMA,
        ],
    )
    def k(idx_hbm, table2_hbm, out_hbm, idx_v, pos_v, bufs, sem_g, sem_w):
        wid = lax.axis_index("s") * NC + lax.axis_index("c")
        base_u = wid * upw
        base_b = wid * bpw
        pltpu.sync_copy(table2_hbm.at[pl.ds(V, S)], pos_v)
        pltpu.sync_copy(idx_hbm.at[pl.ds(base_u * UP, upw * UP)], idx_v)

        def gather_desc(t, buf, half):
            return pltpu.make_async_copy(
                table2_hbm.at[idx_v.at[pl.ds(t * UP, U)]],
                buf.at[pl.ds(half * U, U)],
                sem_g,
            )

        def out_desc(i, buf):
            return pltpu.make_async_copy(
                buf, out_hbm.at[pl.ds((base_b + i) * S, S)], sem_w
            )

        def add_pos(buf, phase):
            def rbody(r4, c2):
                for dr in range(4):
                    r = phase + r4 * 4 + dr
                    for c in range(E // 16):
                        plsc.addupdate(
                            buf.at[r, pl.ds(c * 16, 16)], pos_v[r, pl.ds(c * 16, 16)]
                        )
                return c2

            lax.fori_loop(0, U // 4, rbody, 0)

        def step(t, bi, h, h3, fire, guard_recycle):
            buf = bufs[h3]
            gather_desc(t, buf, h).wait()
            if fire:
                nb3 = (h3 + (h + 3) // 2) % NBB
                nh = (h + 3) % 2
                if nh == 0:
                    rec_i = bi + (h + 3) // 2 - NBB
                    if guard_recycle:

                        @pl.when(rec_i >= 0)
                        def _():
                            out_desc(rec_i, bufs[nb3]).wait()

                    else:
                        out_desc(rec_i, bufs[nb3]).wait()
                gather_desc(t + 3, bufs[nb3], nh).start()
            add_pos(buf, h * U)
            if h == 1:
                out_desc(bi, buf).start()

        gather_desc(0, bufs[0], 0).start()
        gather_desc(1, bufs[0], 1).start()
        gather_desc(2, bufs[1], 0).start()

        def body(j, carry):
            for hh in range(6):
                step(
                    j * 6 + hh,
                    bi=j * 3 + hh // 2,
                    h=hh % 2,
                    h3=(hh // 2) % NBB,
                    fire=True,
                    guard_recycle=True,
                )
            return carry

        nmain = (T - 4) // 6
        lax.fori_loop(0, nmain, body, 0)
        for t in range(nmain * 6, T):
            step(
                t,
                bi=t // 2,
                h=t % 2,
                h3=(t // 2) % NBB,
                fire=(t + 3 < T),
                guard_recycle=False,
            )
        for i in range(bpw - NBB, bpw):
            out_desc(i, bufs[i % NBB]).wait()

    idx_pad = jnp.pad(idx.reshape(nu, U), ((0, 0), (0, UP - U))).reshape(nu * UP)
    return k(idx_pad, table2)


def kernel(pre_embedding, preembed_mask, embed_table, W, b):
    B, S = pre_embedding.shape
    V, E = embed_table.shape
    pe = jnp.asarray(_sinusoidal_pe(S, E))

    table2 = _transform(embed_table, pe, W, b)
    idx = pre_embedding.astype(jnp.int32)
    out = _sc_gather(idx, table2, V, B, S, E)
    return (out.reshape(B, S, E), preembed_mask)


# speedup vs baseline: 1.8926x; 1.0172x over previous
import functools

import numpy as np
import jax
import jax.numpy as jnp
from jax import lax
from jax.experimental import pallas as pl
from jax.experimental.pallas import tpu as pltpu
from jax.experimental.pallas import tpu_sc as plsc


def _sinusoidal_pe(seq_len, d_model):
    pos = np.arange(seq_len, dtype=np.float32)[:, None]
    div = np.exp(np.arange(0, d_model, 2, dtype=np.float32) * (-np.log(10000.0) / d_model))
    pe = np.zeros((seq_len, d_model), dtype=np.float32)
    pe[:, 0::2] = np.sin(pos * div)
    pe[:, 1::2] = np.cos(pos * div)
    return pe


def _transform(table, pe, w, b):
    v, e = table.shape
    s = pe.shape[0]
    bm = 2000
    assert v % bm == 0
    nblk = v // bm

    def body(x_ref, pe_ref, w_ref, b_ref, o_ref):
        pid = pl.program_id(0)

        @pl.when(pid < nblk)
        def _():
            o_ref[...] = jnp.dot(
                x_ref[...], w_ref[:e, :], preferred_element_type=jnp.float32
            )

        @pl.when(pid == nblk)
        def _():
            o_ref[:s, :] = (
                jnp.dot(pe_ref[...], w_ref[e:, :], preferred_element_type=jnp.float32)
                + b_ref[...]
            )

    return pl.pallas_call(
        body,
        grid=(nblk + 1,),
        in_specs=[
            pl.BlockSpec((bm, e), lambda i: (jnp.minimum(i, nblk - 1), 0)),
            pl.BlockSpec((s, e), lambda i: (0, 0)),
            pl.BlockSpec((2 * e, e), lambda i: (0, 0)),
            pl.BlockSpec((1, e), lambda i: (0, 0)),
        ],
        out_specs=pl.BlockSpec((bm, e), lambda i: (i, 0)),
        out_shape=jax.ShapeDtypeStruct((v + bm, e), jnp.float32),
    )(table, pe, w, b.reshape(1, e))


def _sc_gather(idx, table2, V, B, S, E):
    info = plsc.get_sparse_core_info()
    NC, NS = info.num_cores, info.num_subcores
    NW = NC * NS
    U0 = 104
    U1 = S - U0
    NBB = 3
    bpw = B // NW
    T = 2 * bpw

    mesh = plsc.VectorSubcoreMesh(core_axis_name="c", subcore_axis_name="s")

    @functools.partial(
        pl.kernel,
        mesh=mesh,
        out_type=jax.ShapeDtypeStruct((B * S, E), jnp.float32),
        scratch_types=[
            pltpu.VMEM((bpw * S,), jnp.int32),
            pltpu.VMEM((S, E), jnp.float32),
            [pltpu.VMEM((S, E), jnp.float32)] * 3,
            pltpu.SemaphoreType.DMA,
            pltpu.SemaphoreType.DMA,
        ],
    )
    def k(idx_hbm, table2_hbm, out_hbm, idx_v, pos_v, bufs, sem_g, sem_w):
        wid = lax.axis_index("s") * NC + lax.axis_index("c")
        base_b = wid * bpw
        pltpu.sync_copy(table2_hbm.at[pl.ds(V, S)], pos_v)
        pltpu.sync_copy(idx_hbm.at[pl.ds(base_b * S, bpw * S)], idx_v)

        def halves(t, h):
            off = h * U0
            ln = U0 if h == 0 else U1
            return off, ln

        def gather_desc(bi, h, buf):
            off, ln = halves(bi, h)
            return pltpu.make_async_copy(
                table2_hbm.at[idx_v.at[pl.ds(bi * S + off, ln)]],
                buf.at[pl.ds(off, ln)],
                sem_g,
            )

        def out_desc(bi, h, buf):
            off, ln = halves(bi, h)
            return pltpu.make_async_copy(
                buf.at[pl.ds(off, ln)],
                out_hbm.at[pl.ds((base_b + bi) * S + off, ln)],
                sem_w,
            )

        def add_pos(buf, h):
            off, ln = halves(0, h)

            def rbody(r4, c2):
                for dr in range(4):
                    r = off + r4 * 4 + dr
                    for c in range(E // 16):
                        plsc.addupdate(
                            buf.at[r, pl.ds(c * 16, 16)], pos_v[r, pl.ds(c * 16, 16)]
                        )
                return c2

            lax.fori_loop(0, ln // 4, rbody, 0)

        def step(bi, h, h3, fire, guard_recycle):
            buf = bufs[h3]
            gather_desc(bi, h, buf).wait()
            if fire:
                nbi_d = (h + 3) // 2
                nh = (h + 3) % 2
                nb3 = (h3 + nbi_d) % NBB
                rbi_d = (h - 3) // 2
                rh = (h - 3) % 2
                rec = bi + rbi_d
                if guard_recycle:

                    @pl.when(rec >= 0)
                    def _():
                        out_desc(rec, rh, bufs[nb3]).wait()

                else:
                    out_desc(rec, rh, bufs[nb3]).wait()
                gather_desc(bi + nbi_d, nh, bufs[nb3]).start()
            add_pos(buf, h)
            out_desc(bi, h, buf).start()

        gather_desc(0, 0, bufs[0]).start()
        gather_desc(0, 1, bufs[0]).start()
        gather_desc(1, 0, bufs[1]).start()

        def body(j, carry):
            for hh in range(6):
                step(
                    j * 3 + hh // 2,
                    h=hh % 2,
                    h3=(hh // 2) % NBB,
                    fire=True,
                    guard_recycle=True,
                )
            return carry

        nmain = (T - 4) // 6
        lax.fori_loop(0, nmain, body, 0)
        for t in range(nmain * 6, T):
            step(
                t // 2,
                h=t % 2,
                h3=(t // 2) % NBB,
                fire=(t + 3 < T),
                guard_recycle=False,
            )
        for t in range(T - 6, T):
            out_desc(t // 2, t % 2, bufs[(t // 2) % NBB]).wait()

    return k(idx.reshape(B * S), table2)


def kernel(pre_embedding, preembed_mask, embed_table, W, b):
    B, S = pre_embedding.shape
    V, E = embed_table.shape
    pe = jnp.asarray(_sinusoidal_pe(S, E))

    table2 = _transform(embed_table, pe, W, b)
    idx = pre_embedding.astype(jnp.int32)
    out = _sc_gather(idx, table2, V, B, S, E)
    return (out.reshape(B, S, E), preembed_mask)
